# baseline (speedup 1.0000x reference)
import functools

import jax
import jax.numpy as jnp
from jax import lax
from jax.experimental import pallas as pl
from jax.experimental.pallas import tpu as pltpu
from jax.experimental.pallas import tpu_sc as plsc

_L = 16


def _make_sc_kernel(N, F, D, n_workers, rows_per_w, ch):
    n_chunks = rows_per_w // ch
    mesh = plsc.VectorSubcoreMesh(core_axis_name="c", subcore_axis_name="s")

    @functools.partial(
        pl.kernel,
        mesh=mesh,
        compiler_params=pltpu.CompilerParams(
            use_tc_tiling_on_sc=False, needs_layout_passes=False
        ),
        out_type=jax.ShapeDtypeStruct((N, F + 2 * D), jnp.float32),
        scratch_types=[
            pltpu.VMEM((ch, 2), jnp.int32),
            pltpu.VMEM((ch,), jnp.int32),
            pltpu.VMEM((ch,), jnp.int32),
            pltpu.VMEM((ch, _L), jnp.float32),
            pltpu.VMEM((ch, _L), jnp.float32),
            pltpu.SemaphoreType.DMA,
            pltpu.SemaphoreType.DMA,
        ],
    )
    def k(inp_hbm, idx_hbm, wday_hbm, wtime_hbm, out_hbm,
          idxp, didx, tidx, dbuf, tbuf, sem_inp, sem_g):
        wid = lax.axis_index("s") * 2 + lax.axis_index("c")
        base = wid * rows_per_w

        inp_cp = pltpu.make_async_copy(
            inp_hbm.at[pl.ds(base, rows_per_w), :],
            out_hbm.at[pl.ds(base, rows_per_w), pl.ds(0, F)],
            sem_inp,
        )
        inp_cp.start()

        def chunk_body(ci, carry):
            cbase = base + ci * ch
            pltpu.sync_copy(idx_hbm.at[pl.ds(cbase, ch), :], idxp)

            zero = jnp.zeros((_L,), jnp.int32)
            one = jnp.ones((_L,), jnp.int32)

            def deint(i, c):
                r = lax.iota(jnp.int32, _L) + i * _L
                d = plsc.load_gather(idxp, [r, zero])
                t = plsc.load_gather(idxp, [r, one])
                didx[pl.ds(i * _L, _L)] = d
                tidx[pl.ds(i * _L, _L)] = t
                return c

            lax.fori_loop(0, ch // _L, deint, 0)

            gd = pltpu.make_async_copy(wday_hbm.at[didx], dbuf, sem_g)
            gd.start()
            gt = pltpu.make_async_copy(wtime_hbm.at[tidx], tbuf, sem_g)
            gt.start()
            gd.wait()
            gt.wait()
            pltpu.sync_copy(dbuf, out_hbm.at[pl.ds(cbase, ch), pl.ds(F, D)])
            pltpu.sync_copy(tbuf, out_hbm.at[pl.ds(cbase, ch), pl.ds(F + D, D)])
            return carry

        lax.fori_loop(0, n_chunks, chunk_body, 0)
        inp_cp.wait()

    return k


def kernel(inp, daytime, W_day, W_time):
    B, T, F = inp.shape
    D = W_day.shape[1]
    N = B * T
    n_workers = 32
    rows_per_w = N // n_workers
    ch = 2560
    assert rows_per_w * n_workers == N and rows_per_w % ch == 0

    inp2 = inp.reshape(N, F)
    idx2 = daytime.reshape(N, 2).astype(jnp.int32)
    k = _make_sc_kernel(N, F, D, n_workers, rows_per_w, ch)
    out = k(inp2, idx2, W_day, W_time)
    return out.reshape(B, T, F + 2 * D)


# speedup vs baseline: 2.5444x; 2.5444x over previous
import functools

import jax
import jax.numpy as jnp
from jax import lax
from jax.experimental import pallas as pl
from jax.experimental.pallas import tpu as pltpu
from jax.experimental.pallas import tpu_sc as plsc

_L = 16


def _make_sc_kernel(N, F, D, n_workers, rows_per_w, ch):
    n_chunks = rows_per_w // ch
    W = F + 2 * D
    mesh = plsc.VectorSubcoreMesh(core_axis_name="c", subcore_axis_name="s")

    @functools.partial(
        pl.kernel,
        mesh=mesh,
        compiler_params=pltpu.CompilerParams(
            use_tc_tiling_on_sc=False, needs_layout_passes=False
        ),
        out_type=jax.ShapeDtypeStruct((N, W), jnp.float32),
        scratch_types=[
            pltpu.VMEM((ch, W), jnp.float32),
            pltpu.VMEM((ch, 2), jnp.int32),
            pltpu.VMEM((8, _L), jnp.float32),
            pltpu.VMEM((288, _L), jnp.float32),
            pltpu.SemaphoreType.DMA,
        ],
    )
    def k(inp_hbm, idx_hbm, wday_hbm, wtime_hbm, out_hbm,
          rows, idxp, wday_v, wtime_v, sem_inp):
        wid = lax.axis_index("s") * 2 + lax.axis_index("c")
        base = wid * rows_per_w

        pltpu.sync_copy(wday_hbm, wday_v.at[pl.ds(0, 7), :])
        pltpu.sync_copy(wtime_hbm, wtime_v)

        iota = lax.iota(jnp.int32, _L)
        zero = jnp.zeros((_L,), jnp.int32)
        one = jnp.ones((_L,), jnp.int32)

        def chunk_body(ci, carry):
            cbase = base + ci * ch
            inp_cp = pltpu.make_async_copy(
                inp_hbm.at[pl.ds(cbase, ch), :],
                rows.at[:, pl.ds(0, F)],
                sem_inp,
            )
            inp_cp.start()
            pltpu.sync_copy(idx_hbm.at[pl.ds(cbase, ch), :], idxp)

            def group_body(g, c):
                r = iota + g * _L
                d = plsc.load_gather(idxp, [r, zero])
                t = plsc.load_gather(idxp, [r, one])
                for col in range(D):
                    cvec = jnp.full((_L,), col, jnp.int32)
                    vd = plsc.load_gather(wday_v, [d, cvec])
                    plsc.store_scatter(
                        rows, [r, jnp.full((_L,), F + col, jnp.int32)], vd)
                    vt = plsc.load_gather(wtime_v, [t, cvec])
                    plsc.store_scatter(
                        rows, [r, jnp.full((_L,), F + D + col, jnp.int32)], vt)
                return c

            lax.fori_loop(0, ch // _L, group_body, 0)
            inp_cp.wait()
            pltpu.sync_copy(rows, out_hbm.at[pl.ds(cbase, ch), :])
            return carry

        lax.fori_loop(0, n_chunks, chunk_body, 0)

    return k


def kernel(inp, daytime, W_day, W_time):
    B, T, F = inp.shape
    D = W_day.shape[1]
    N = B * T
    n_workers = 32
    rows_per_w = N // n_workers
    ch = 1024
    assert rows_per_w * n_workers == N and rows_per_w % ch == 0

    inp2 = inp.reshape(N, F)
    idx2 = daytime.reshape(N, 2).astype(jnp.int32)
    k = _make_sc_kernel(N, F, D, n_workers, rows_per_w, ch)
    out = k(inp2, idx2, W_day, W_time)
    return out.reshape(B, T, F + 2 * D)
